# Initial kernel scaffold; baseline (speedup 1.0000x reference)
import jax
import jax.numpy as jnp
from jax.experimental import pallas as pl


def kernel(x, w1a, b1a, w1b, b1b, w1c, b1c, w2a, b2a, w2b, b2b, w2c, b2c, wg, bg, we1, be1, we2, be2, we3, be3):
    raise NotImplementedError("write your pallas kernel here")


# speedup vs baseline: 143.7526x; 143.7526x over previous
import functools

import jax
import jax.numpy as jnp
from jax import lax
from jax.experimental import pallas as pl
from jax.experimental.pallas import tpu as pltpu
from jax.experimental.pallas import tpu_sc as plsc

_B, _N, _K, _OUT = 4, 2048, 20, 13
_EPS = 1e-5

_pcall = pl.pallas_call


def _knn_body(xp_ref, x2c_ref, x2r_ref, out_ref):
    xp = xp_ref[0]
    dot = lax.dot_general(xp, xp, (((1,), (1,)), ((), ())),
                          preferred_element_type=jnp.float32)
    d = (x2c_ref[0] + x2r_ref[0]) - 2.0 * dot
    iota = lax.broadcasted_iota(jnp.int32, (_N, _N), 1).astype(jnp.float32)
    cols = []
    for j in range(_K + 1):
        m = jnp.min(d, axis=1, keepdims=True)
        cand = jnp.where(d == m, iota, float(_N))
        a = jnp.min(cand, axis=1, keepdims=True)
        if j > 0:
            cols.append(a.astype(jnp.int32))
        d = jnp.where(iota == a, jnp.inf, d)
    out_ref[0] = jnp.concatenate(cols, axis=1)


def _knn(x_pad, x2):
    return _pcall(
        _knn_body,
        grid=(_B,),
        in_specs=[
            pl.BlockSpec((1, _N, 8), lambda b: (b, 0, 0)),
            pl.BlockSpec((1, _N, 1), lambda b: (b, 0, 0)),
            pl.BlockSpec((1, 1, _N), lambda b: (b, 0, 0)),
        ],
        out_specs=pl.BlockSpec((1, _N, _K), lambda b: (b, 0, 0)),
        out_shape=jax.ShapeDtypeStruct((_B, _N, _K), jnp.int32),
    )(x_pad, x2.reshape(_B, _N, 1), x2.reshape(_B, 1, _N))


def _bnrelu_body(stats_ref, y_ref, o_ref):
    cnt = jnp.float32(_B * _N)
    m = stats_ref[0:1, :] / cnt
    ey2 = stats_ref[1:2, :] / cnt
    rs = lax.rsqrt(ey2 - m * m + _EPS)
    o_ref[0] = jnp.maximum((y_ref[0] - m) * rs, 0.0)


def _bnrelu(y, stats, c):
    return _pcall(
        _bnrelu_body,
        grid=(_B,),
        in_specs=[
            pl.BlockSpec((8, c), lambda b: (0, 0)),
            pl.BlockSpec((1, _N, c), lambda b: (b, 0, 0)),
        ],
        out_specs=pl.BlockSpec((1, _N, c), lambda b: (b, 0, 0)),
        out_shape=jax.ShapeDtypeStruct((_B, _N, c), jnp.float32),
    )(stats, y)


def _gather_rows(table, idxg, chunk):
    M = idxg.shape[0]
    D = table.shape[1]
    nw = 32
    b_per_w = M // nw
    n_chunks = b_per_w // chunk
    mesh = plsc.VectorSubcoreMesh(core_axis_name="c", subcore_axis_name="s")

    @functools.partial(
        pl.kernel,
        mesh=mesh,
        out_type=jax.ShapeDtypeStruct((M, D), jnp.float32),
        scratch_types=[
            pltpu.VMEM((chunk,), jnp.int32),
            pltpu.VMEM((chunk, D), jnp.float32),
            pltpu.SemaphoreType.DMA,
        ],
    )
    def k(table_hbm, idx_hbm, out_hbm, idx_v, rows_v, sem):
        wid = lax.axis_index("s") * 2 + lax.axis_index("c")
        base = wid * b_per_w

        def body(i, carry):
            off = base + i * chunk
            pltpu.sync_copy(idx_hbm.at[pl.ds(off, chunk)], idx_v)
            pltpu.async_copy(table_hbm.at[idx_v], rows_v, sem).wait()
            pltpu.sync_copy(rows_v, out_hbm.at[pl.ds(off, chunk)])
            return carry

        lax.fori_loop(0, n_chunks, body, 0)

    return k(table, idxg)


def _pass_a_body(pf_ref, g_ref, gr_ref, w_ref, y_ref, st_ref):
    b = pl.program_id(0)
    k = pl.program_id(1)

    @pl.when(jnp.logical_and(b == 0, k == 0))
    def _():
        st_ref[...] = jnp.zeros_like(st_ref)

    pf = pf_ref[0]
    c = pf.shape[1]
    rf = jnp.concatenate(
        [pf, g_ref[0, 0][:, :c] - pf, gr_ref[0, 0][:, :c] - pf],
        axis=1)
    y = jnp.dot(rf, w_ref[...], preferred_element_type=jnp.float32)
    y_ref[0, 0] = y
    st_ref[0:1, :] += jnp.sum(y, axis=0, keepdims=True)
    st_ref[1:2, :] += jnp.sum(y * y, axis=0, keepdims=True)


def _pass_a(pf, g, wt, c_in, c_out):
    return _pcall(
        _pass_a_body,
        grid=(_B, _K),
        in_specs=[
            pl.BlockSpec((1, _N, c_in), lambda b, k: (b, 0, 0)),
            pl.BlockSpec((1, 1, _N, 128), lambda b, k: (b, k, 0, 0)),
            pl.BlockSpec((1, 1, _N, 128), lambda b, k: (b, (k + 1) % _K, 0, 0)),
            pl.BlockSpec((3 * c_in, c_out), lambda b, k: (0, 0)),
        ],
        out_specs=[
            pl.BlockSpec((1, 1, _N, c_out), lambda b, k: (b, k, 0, 0)),
            pl.BlockSpec((8, c_out), lambda b, k: (0, 0)),
        ],
        out_shape=[
            jax.ShapeDtypeStruct((_B, _K, _N, c_out), jnp.float32),
            jax.ShapeDtypeStruct((8, c_out), jnp.float32),
        ],
        compiler_params=pltpu.CompilerParams(
            dimension_semantics=("arbitrary", "arbitrary")),
    )(pf, g, g, wt)


def _pass_b_body(st_ref, y_ref, w_ref, z_ref, st2_ref, *, cnt):
    b = pl.program_id(0)
    k = pl.program_id(1)

    @pl.when(jnp.logical_and(b == 0, k == 0))
    def _():
        st2_ref[...] = jnp.zeros_like(st2_ref)

    m = st_ref[0:1, :] / cnt
    ey2 = st_ref[1:2, :] / cnt
    rs = lax.rsqrt(ey2 - m * m + _EPS)
    h = jnp.maximum((y_ref[0, 0] - m) * rs, 0.0)
    z = jnp.dot(h, w_ref[...], preferred_element_type=jnp.float32)
    z_ref[0, 0] = z
    st2_ref[0:1, :] += jnp.sum(z, axis=0, keepdims=True)
    st2_ref[1:2, :] += jnp.sum(z * z, axis=0, keepdims=True)


def _pass_b(y, stats, wt, c):
    return _pcall(
        functools.partial(_pass_b_body, cnt=float(_B * _K * _N)),
        grid=(_B, _K),
        in_specs=[
            pl.BlockSpec((8, c), lambda b, k: (0, 0)),
            pl.BlockSpec((1, 1, _N, c), lambda b, k: (b, k, 0, 0)),
            pl.BlockSpec((c, c), lambda b, k: (0, 0)),
        ],
        out_specs=[
            pl.BlockSpec((1, 1, _N, c), lambda b, k: (b, k, 0, 0)),
            pl.BlockSpec((8, c), lambda b, k: (0, 0)),
        ],
        out_shape=[
            jax.ShapeDtypeStruct((_B, _K, _N, c), jnp.float32),
            jax.ShapeDtypeStruct((8, c), jnp.float32),
        ],
        compiler_params=pltpu.CompilerParams(
            dimension_semantics=("arbitrary", "arbitrary")),
    )(stats, y, wt)


def _pass_c_body(st_ref, z_ref, w_ref, yc_ref, st2_ref, *, cnt):
    b = pl.program_id(0)

    @pl.when(b == 0)
    def _():
        st2_ref[...] = jnp.zeros_like(st2_ref)

    m = st_ref[0:1, :] / cnt
    ey2 = st_ref[1:2, :] / cnt
    rs = lax.rsqrt(ey2 - m * m + _EPS)
    acc = jnp.maximum((z_ref[0, 0] - m) * rs, 0.0)
    for k in range(1, _K):
        acc = jnp.maximum(acc, jnp.maximum((z_ref[0, k] - m) * rs, 0.0))
    yc = jnp.dot(acc, w_ref[...], preferred_element_type=jnp.float32)
    yc_ref[0] = yc
    st2_ref[0:1, :] += jnp.sum(yc, axis=0, keepdims=True)
    st2_ref[1:2, :] += jnp.sum(yc * yc, axis=0, keepdims=True)


def _pass_c(z, stats, wt, c):
    return _pcall(
        functools.partial(_pass_c_body, cnt=float(_B * _K * _N)),
        grid=(_B,),
        in_specs=[
            pl.BlockSpec((8, c), lambda b: (0, 0)),
            pl.BlockSpec((1, _K, _N, c), lambda b: (b, 0, 0, 0)),
            pl.BlockSpec((c, c), lambda b: (0, 0)),
        ],
        out_specs=[
            pl.BlockSpec((1, _N, c), lambda b: (b, 0, 0)),
            pl.BlockSpec((8, c), lambda b: (0, 0)),
        ],
        out_shape=[
            jax.ShapeDtypeStruct((_B, _N, c), jnp.float32),
            jax.ShapeDtypeStruct((8, c), jnp.float32),
        ],
        compiler_params=pltpu.CompilerParams(
            dimension_semantics=("arbitrary",)),
    )(stats, z, wt)


def _head_body(o1_ref, yc2_ref, st2_ref, wgt_ref, bg_ref, w1t_ref, w2t_ref,
               w3t_ref, out_ref, e1_ref):
    cnt = jnp.float32(_B * _N)
    m = st2_ref[0:1, :] / cnt
    ey2 = st2_ref[1:2, :] / cnt
    rs = lax.rsqrt(ey2 - m * m + _EPS)

    s1 = jnp.zeros((1, 256), jnp.float32)
    q1 = jnp.zeros((1, 256), jnp.float32)
    for b in range(_B):
        o1b = o1_ref[b]
        o2b = jnp.maximum((yc2_ref[b] - m) * rs, 0.0)
        gi = jnp.concatenate([o1b, o2b], axis=1)
        go = jnp.dot(gi, wgt_ref[...],
                     preferred_element_type=jnp.float32) + bg_ref[0:1, :]
        gmax = jnp.max(go, axis=0, keepdims=True)
        cc = jnp.concatenate(
            [jnp.broadcast_to(gmax, (_N, 1024)), gi], axis=1)
        f1 = jnp.dot(cc, w1t_ref[...], preferred_element_type=jnp.float32)
        e1_ref[pl.ds(b * _N, _N), :] = f1
        s1 = s1 + jnp.sum(f1, axis=0, keepdims=True)
        q1 = q1 + jnp.sum(f1 * f1, axis=0, keepdims=True)

    m1 = s1 / cnt
    rs1 = lax.rsqrt(q1 / cnt - m1 * m1 + _EPS)
    h1 = jnp.maximum((e1_ref[...] - m1) * rs1, 0.0)
    f2 = jnp.dot(h1, w2t_ref[...], preferred_element_type=jnp.float32)
    m2 = jnp.mean(f2, axis=0, keepdims=True)
    rs2 = lax.rsqrt(jnp.mean(f2 * f2, axis=0, keepdims=True) - m2 * m2 + _EPS)
    h2 = jnp.maximum((f2 - m2) * rs2, 0.0)
    f3 = jnp.dot(h2, w3t_ref[...], preferred_element_type=jnp.float32)
    m3 = jnp.mean(f3, axis=0, keepdims=True)
    rs3 = lax.rsqrt(jnp.mean(f3 * f3, axis=0, keepdims=True) - m3 * m3 + _EPS)
    out_ref[...] = (f3 - m3) * rs3


def _head(o1, yc2, st2, wgt, bg2, w1t, w2t, w3t):
    return _pcall(
        _head_body,
        in_specs=[
            pl.BlockSpec(o1.shape, lambda: (0, 0, 0)),
            pl.BlockSpec(yc2.shape, lambda: (0, 0, 0)),
            pl.BlockSpec(st2.shape, lambda: (0, 0)),
            pl.BlockSpec(wgt.shape, lambda: (0, 0)),
            pl.BlockSpec(bg2.shape, lambda: (0, 0)),
            pl.BlockSpec(w1t.shape, lambda: (0, 0)),
            pl.BlockSpec(w2t.shape, lambda: (0, 0)),
            pl.BlockSpec(w3t.shape, lambda: (0, 0)),
        ],
        out_specs=pl.BlockSpec((_B * _N, _OUT), lambda: (0, 0)),
        out_shape=jax.ShapeDtypeStruct((_B * _N, _OUT), jnp.float32),
        scratch_shapes=[pltpu.VMEM((_B * _N, 256), jnp.float32)],
    )(o1, yc2, st2, wgt, bg2, w1t, w2t, w3t)


def _rrl(pf, table, idxg, wat, wbt, wct, c_in, c_out, chunk):
    g = _gather_rows(table, idxg, chunk).reshape(_B, _K, _N, 128)
    y, st_a = _pass_a(pf, g, wat, c_in, c_out)
    z, st_b = _pass_b(y, st_a, wbt, c_out)
    return _pass_c(z, st_b, wct, c_out)


def kernel(x, w1a, b1a, w1b, b1b, w1c, b1c, w2a, b2a, w2b, b2b, w2c, b2c,
           wg, bg, we1, be1, we2, be2, we3, be3):
    x_pad = jnp.pad(x, ((0, 0), (0, 0), (0, 5)))
    x2 = jnp.sum(x * x, axis=-1)
    nn = _knn(x_pad, x2)
    gidx = (nn + (jnp.arange(_B, dtype=jnp.int32) * _N)[:, None, None])
    idxg = jnp.transpose(gidx, (0, 2, 1)).reshape(-1)

    x16 = jnp.pad(x, ((0, 0), (0, 0), (0, 13)))
    z3 = jnp.zeros((13, 64), jnp.float32)
    wt1 = jnp.concatenate([w1a[:, 0:3].T, z3, w1a[:, 3:6].T, z3,
                           w1a[:, 6:9].T, z3], axis=0)
    t1 = jnp.pad(x, ((0, 0), (0, 0), (0, 125))).reshape(_B * _N, 128)
    yc1, st_c1 = _rrl(x16, t1, idxg, wt1, w1b.T, w1c.T, 16, 64, 256)

    o1 = _bnrelu(yc1, st_c1, 64)
    t2 = jnp.pad(o1, ((0, 0), (0, 0), (0, 64))).reshape(_B * _N, 128)
    yc2, st_c2 = _rrl(o1, t2, idxg, w2a.T, w2b.T, w2c.T, 64, 128, 256)

    out = _head(o1, yc2, st_c2, wg.T, bg.reshape(1, -1),
                we1.T, we2.T, we3.T)
    return jnp.transpose(out.reshape(_B, _N, _OUT), (0, 2, 1))
